# baseline (speedup 1.0000x reference)
import functools

import jax
import jax.numpy as jnp
from jax import lax
from jax.experimental import pallas as pl
from jax.experimental.pallas import tpu as pltpu
from jax.experimental.pallas import tpu_sc as plsc

_D = 59
_OUT_C = 64
_X = 128
_Y = 128
_Z = 7
_STRIDE = 16.0
_NV = _X * _Y * _Z
_NCOL = _X * _Y
_NW = 32
_CPW = _NCOL // _NW
_FPSTR = _OUT_C // 2 + 1
_FPSZ = 704 * _FPSTR
_DPSTR = _D // 2 + 1 + 1
_DPZERO = 704 * _DPSTR
_DPSZ = _DPZERO + 8


def _dense_body(x_ref, w_ref, b_ref, depth_ref, feat_ref):
    xb = x_ref[0]
    w = w_ref[...]
    out = jax.lax.dot_general(xb.astype(jnp.bfloat16), w.astype(jnp.bfloat16),
                              (((0,), (1,)), ((), ())),
                              preferred_element_type=jnp.float32)
    out = out + b_ref[...]
    logits = out[:, :_D]
    m = jnp.max(logits, axis=1, keepdims=True)
    e = jnp.exp(logits - m)
    s = jnp.sum(e, axis=1, keepdims=True)
    depth_ref[0] = e / s
    feat_ref[0] = out[:, _D:_D + _OUT_C]


def _dense_stage(img_feats, W_dn, b_dn):
    B, N, C, H, W = img_feats.shape
    BN = B * N
    HW = H * W
    x = img_feats.reshape(BN, C, HW)
    w_p = jnp.zeros((128, C), jnp.float32).at[:_D + _OUT_C].set(W_dn)
    b_p = jnp.zeros((1, 128), jnp.float32).at[0, :_D + _OUT_C].set(b_dn)
    depth, feat = pl.pallas_call(
        _dense_body,
        grid=(BN,),
        in_specs=[
            pl.BlockSpec((1, C, HW), lambda i: (i, 0, 0)),
            pl.BlockSpec((128, C), lambda i: (0, 0)),
            pl.BlockSpec((1, 128), lambda i: (0, 0)),
        ],
        out_specs=[
            pl.BlockSpec((1, HW, _D), lambda i: (i, 0, 0)),
            pl.BlockSpec((1, HW, _OUT_C), lambda i: (i, 0, 0)),
        ],
        out_shape=[
            jax.ShapeDtypeStruct((BN, HW, _D), jnp.float32),
            jax.ShapeDtypeStruct((BN, HW, _OUT_C), jnp.float32),
        ],
    )(x, w_p, b_p)
    return depth, feat


_sc_mesh = plsc.VectorSubcoreMesh(core_axis_name="c", subcore_axis_name="s")


@functools.partial(
    pl.kernel,
    out_type=jax.ShapeDtypeStruct((2, _OUT_C, _NW, _CPW), jnp.float32),
    mesh=_sc_mesh,
    compiler_params=pltpu.CompilerParams(needs_layout_passes=False),
    scratch_types=[
        pltpu.VMEM((_FPSZ,), jnp.int32),
        pltpu.VMEM((_FPSZ,), jnp.int32),
        pltpu.VMEM((_DPSZ,), jnp.int32),
        pltpu.VMEM((_DPSZ,), jnp.int32),
        pltpu.VMEM((_Z * _CPW,), jnp.int32),
        pltpu.VMEM((_Z * _CPW,), jnp.int32),
        pltpu.VMEM((_OUT_C, _CPW), jnp.float32),
        pltpu.SemaphoreType.DMA,
        pltpu.SemaphoreType.DMA,
    ],
)
def _sc_gather(feat_hbm, dpt_hbm, idx_hbm, out_hbm,
               feat_v0, feat_v1, dpt_v0, dpt_v1, idx_v0, idx_v1, acc_v,
               sem0, sem1):
    wid = lax.axis_index("s") * 2 + lax.axis_index("c")
    zero16 = jnp.zeros((16,), jnp.float32)
    m_lo15 = jnp.int32(0x7FFF)
    m_par = jnp.int32(0x8000)
    m_hi16 = jnp.int32(-65536)
    feat_b = (feat_v0, feat_v1)
    dpt_b = (dpt_v0, dpt_v1)
    idx_b = (idx_v0, idx_v1)
    sem_b = (sem0, sem1)

    def _start(bn, p):
        pltpu.async_copy(feat_hbm.at[bn], feat_b[p], sem_b[p])
        pltpu.async_copy(dpt_hbm.at[bn], dpt_b[p], sem_b[p])
        pltpu.async_copy(idx_hbm.at[bn, wid], idx_b[p], sem_b[p])

    def _wait(p):
        pltpu.make_async_copy(feat_hbm.at[0], feat_b[p], sem_b[p]).wait()
        pltpu.make_async_copy(dpt_hbm.at[0], dpt_b[p], sem_b[p]).wait()
        pltpu.make_async_copy(idx_hbm.at[0, 0], idx_b[p], sem_b[p]).wait()

    _start(0, 0)

    def _mbody(m, _):
        for p in range(2):
            bn = 2 * m + p
            _wait(p)

            @pl.when(bn + 1 < 12)
            def _():
                _start(bn + 1, 1 - p)

            @pl.when((bn == 0) | (bn == 6))
            def _():
                def _zero(i, _):
                    def _zg(g, _):
                        acc_v[i, pl.ds(g * 16, 16)] = zero16
                        return None
                    lax.fori_loop(0, _CPW // 16, _zg, None)
                    return None
                lax.fori_loop(0, _OUT_C, _zero, None)

            feat_v, dpt_v, idx_v = feat_b[p], dpt_b[p], idx_b[p]

            def _zbody(z, _):
                @plsc.parallel_loop(0, _CPW // 16, 1, unroll=2)
                def _gbody(g):
                    base = z * _CPW + g * 16
                    iv = idx_v[pl.ds(base, 16)]
                    didxw = iv & m_lo15
                    par = (iv & m_par) != 0
                    fbw = lax.shift_right_logical(iv, 16)
                    wword = plsc.load_gather(dpt_v, [didxw])
                    w_lo = plsc.bitcast(wword << 16, jnp.float32)
                    w_hi = plsc.bitcast(wword & m_hi16, jnp.float32)
                    wvec = jnp.where(par, w_hi, w_lo)
                    for c in range(_OUT_C // 2):
                        word = plsc.load_gather(feat_v, [fbw + c])
                        f_e = plsc.bitcast(word << 16, jnp.float32)
                        f_o = plsc.bitcast(word & m_hi16, jnp.float32)
                        plsc.addupdate(
                            acc_v.at[2 * c, pl.ds(g * 16, 16)], f_e * wvec)
                        plsc.addupdate(
                            acc_v.at[2 * c + 1, pl.ds(g * 16, 16)], f_o * wvec)
                return None
            lax.fori_loop(0, _Z, _zbody, None)

            @pl.when((bn == 5) | (bn == 11))
            def _():
                pltpu.sync_copy(acc_v, out_hbm.at[bn // 6, :, wid])
        return None
    lax.fori_loop(0, 6, _mbody, None)


def _make_perm():
    import numpy as np
    j = np.arange(_NV)
    t = j // (_Z * _CPW)
    z = (j // _CPW) % _Z
    lcol = j % _CPW
    col = t * _CPW + lcol
    y = col // _X
    x = col % _X
    return x * (_Y * _Z) + y * _Z + z


_PERM = _make_perm()


def _make_vox_table():
    import numpy as np
    xs, ys, zs = np.meshgrid(np.arange(_X), np.arange(_Y), np.arange(_Z),
                             indexing='ij')
    coords = np.stack([xs, ys, zs], axis=3).astype(np.float32)
    lb = np.array([-51.2, -51.2, -2.5], dtype=np.float32)
    iv = np.array([0.8, 0.8, 1.0], dtype=np.float32)
    vc = (coords * iv + lb).reshape(-1, 3)[_PERM]
    return np.concatenate([vc.T, np.ones((1, vc.shape[0]), np.float32)],
                          axis=0)


_VOX_T = _make_vox_table()


def _idx_body(cam_ref, img_ref, idx_ref, H, W):
    z = cam_ref[0, 2]
    fc0 = img_ref[0, 0] / _STRIDE
    fc1 = img_ref[0, 1] / _STRIDE
    valid = ((z > 0.5) & (fc0 >= 0) & (fc0 < W) & (fc1 >= 0) & (fc1 < H))
    depth_bin = (z - 1.0).astype(jnp.int32)
    valid = valid & (depth_bin >= 0) & (depth_bin < _D)
    u = jnp.clip(fc0.astype(jnp.int32), 0, W - 1)
    v = jnp.clip(fc1.astype(jnp.int32), 0, H - 1)
    d = jnp.clip(depth_bin, 0, _D - 1)
    pix = v * W + u
    dpart = jnp.where(valid, (pix * _DPSTR + (d >> 1)) | ((d & 1) << 15),
                      _DPZERO)
    idx_ref[0] = dpart | ((pix * _FPSTR) << 16)


def _projection_idx(cam2ego, cam_intrinsics, H, W):
    vox_homo_t = jnp.asarray(_VOX_T)
    nv = vox_homo_t.shape[1]

    def _proj(e2c_bn, K_bn):
        cam_t = (e2c_bn @ vox_homo_t)[:3]
        z = cam_t[2]
        z_safe = jnp.clip(z, 0.1, None)
        norm2_t = cam_t[:2] / z_safe[None, :]
        homo_t = jnp.concatenate([norm2_t, jnp.ones((1, nv), jnp.float32)],
                                 axis=0)
        img_t = (K_bn @ homo_t)[:2]
        return cam_t, img_t

    e2c = jnp.linalg.inv(cam2ego)
    cam_t, img_t = jax.vmap(jax.vmap(_proj))(e2c, cam_intrinsics)
    BN = cam_t.shape[0] * cam_t.shape[1]
    cam_t = cam_t.reshape(BN, 3, _NV // 128, 128)
    img_t = img_t.reshape(BN, 2, _NV // 128, 128)
    idx = pl.pallas_call(
        functools.partial(_idx_body, H=H, W=W),
        grid=(BN,),
        in_specs=[
            pl.BlockSpec((1, 3, _NV // 128, 128), lambda i: (i, 0, 0, 0)),
            pl.BlockSpec((1, 2, _NV // 128, 128), lambda i: (i, 0, 0, 0)),
        ],
        out_specs=pl.BlockSpec((1, _NV // 128, 128), lambda i: (i, 0, 0)),
        out_shape=jax.ShapeDtypeStruct((BN, _NV // 128, 128), jnp.int32),
    )(cam_t, img_t)
    return idx.reshape(BN, _NV)


def kernel(img_feats, cam2ego, cam_intrinsics, W_dn, b_dn):
    B, N, C, H, W = img_feats.shape
    BN = B * N
    depth, feat = _dense_stage(img_feats, W_dn, b_dn)

    idx_r = _projection_idx(cam2ego, cam_intrinsics, H, W) \
                .reshape(BN, _NW, _Z * _CPW)

    featp = jax.lax.bitcast_convert_type(
        feat.astype(jnp.bfloat16).reshape(BN, 704, _OUT_C // 2, 2), jnp.int32)
    featp = jnp.pad(featp, ((0, 0), (0, 0), (0, 1))).reshape(BN, _FPSZ)
    dptp = jax.lax.bitcast_convert_type(
        jnp.pad(depth.astype(jnp.bfloat16), ((0, 0), (0, 0), (0, 1)))
           .reshape(BN, 704, _DPSTR - 1, 2), jnp.int32)
    dptp = jnp.pad(dptp, ((0, 0), (0, 0), (0, 1))).reshape(BN, _DPZERO)
    dptp = jnp.pad(dptp, ((0, 0), (0, _DPSZ - _DPZERO)))

    out = _sc_gather(featp, dptp, idx_r)
    bev = out.reshape(B, _OUT_C, _Y, _X)

    depth_out = depth.reshape(B, N, H, W, _D)
    return bev, depth_out


# speedup vs baseline: 1.1555x; 1.1555x over previous
import functools

import jax
import jax.numpy as jnp
from jax import lax
from jax.experimental import pallas as pl
from jax.experimental.pallas import tpu as pltpu
from jax.experimental.pallas import tpu_sc as plsc

_D = 59
_OUT_C = 64
_X = 128
_Y = 128
_Z = 7
_STRIDE = 16.0
_NV = _X * _Y * _Z
_NCOL = _X * _Y
_NW = 32
_CPW = _NCOL // _NW
_FPSTR = _OUT_C // 2 + 1
_FPSZ = 704 * _FPSTR
_DPSTR = _D // 2 + 1 + 1
_DPZERO = 704 * _DPSTR
_DPSZ = _DPZERO + 8


def _dense_body(x_ref, w_ref, b_ref, depth_ref, feat_ref):
    xb = x_ref[0]
    w = w_ref[...]
    out = jax.lax.dot_general(xb.astype(jnp.bfloat16), w.astype(jnp.bfloat16),
                              (((0,), (1,)), ((), ())),
                              preferred_element_type=jnp.float32)
    out = out + b_ref[...]
    logits = out[:, :_D]
    m = jnp.max(logits, axis=1, keepdims=True)
    e = jnp.exp(logits - m)
    s = jnp.sum(e, axis=1, keepdims=True)
    depth_ref[0] = e / s
    feat_ref[0] = out[:, _D:_D + _OUT_C]


def _dense_stage(img_feats, W_dn, b_dn):
    B, N, C, H, W = img_feats.shape
    BN = B * N
    HW = H * W
    x = img_feats.reshape(BN, C, HW)
    w_p = jnp.zeros((128, C), jnp.float32).at[:_D + _OUT_C].set(W_dn)
    b_p = jnp.zeros((1, 128), jnp.float32).at[0, :_D + _OUT_C].set(b_dn)
    depth, feat = pl.pallas_call(
        _dense_body,
        grid=(BN,),
        in_specs=[
            pl.BlockSpec((1, C, HW), lambda i: (i, 0, 0)),
            pl.BlockSpec((128, C), lambda i: (0, 0)),
            pl.BlockSpec((1, 128), lambda i: (0, 0)),
        ],
        out_specs=[
            pl.BlockSpec((1, HW, _D), lambda i: (i, 0, 0)),
            pl.BlockSpec((1, HW, _OUT_C), lambda i: (i, 0, 0)),
        ],
        out_shape=[
            jax.ShapeDtypeStruct((BN, HW, _D), jnp.float32),
            jax.ShapeDtypeStruct((BN, HW, _OUT_C), jnp.float32),
        ],
    )(x, w_p, b_p)
    return depth, feat


_sc_mesh = plsc.VectorSubcoreMesh(core_axis_name="c", subcore_axis_name="s")


@functools.partial(
    pl.kernel,
    out_type=jax.ShapeDtypeStruct((2, _OUT_C, _NW, _CPW), jnp.float32),
    mesh=_sc_mesh,
    compiler_params=pltpu.CompilerParams(needs_layout_passes=False),
    scratch_types=[
        pltpu.VMEM((_FPSZ,), jnp.int32),
        pltpu.VMEM((_FPSZ,), jnp.int32),
        pltpu.VMEM((_DPSZ,), jnp.int32),
        pltpu.VMEM((_DPSZ,), jnp.int32),
        pltpu.VMEM((_Z * _CPW,), jnp.int32),
        pltpu.VMEM((_Z * _CPW,), jnp.int32),
        pltpu.VMEM((_OUT_C, _CPW), jnp.float32),
        pltpu.SemaphoreType.DMA,
        pltpu.SemaphoreType.DMA,
    ],
)
def _sc_gather(feat_hbm, dpt_hbm, idx_hbm, out_hbm,
               feat_v0, feat_v1, dpt_v0, dpt_v1, idx_v0, idx_v1, acc_v,
               sem0, sem1):
    wid = lax.axis_index("s") * 2 + lax.axis_index("c")
    zero16 = jnp.zeros((16,), jnp.float32)
    m_lo15 = jnp.int32(0x7FFF)
    m_par = jnp.int32(0x8000)
    m_hi16 = jnp.int32(-65536)
    feat_b = (feat_v0, feat_v1)
    dpt_b = (dpt_v0, dpt_v1)
    idx_b = (idx_v0, idx_v1)
    sem_b = (sem0, sem1)

    def _start(bn, p):
        pltpu.async_copy(feat_hbm.at[bn], feat_b[p], sem_b[p])
        pltpu.async_copy(dpt_hbm.at[bn], dpt_b[p], sem_b[p])
        pltpu.async_copy(idx_hbm.at[bn, wid], idx_b[p], sem_b[p])

    def _wait(p):
        pltpu.make_async_copy(feat_hbm.at[0], feat_b[p], sem_b[p]).wait()
        pltpu.make_async_copy(dpt_hbm.at[0], dpt_b[p], sem_b[p]).wait()
        pltpu.make_async_copy(idx_hbm.at[0, 0], idx_b[p], sem_b[p]).wait()

    _start(0, 0)

    def _mbody(m, _):
        for p in range(2):
            bn = 2 * m + p
            _wait(p)

            @pl.when(bn + 1 < 12)
            def _():
                _start(bn + 1, 1 - p)

            @pl.when((bn == 0) | (bn == 6))
            def _():
                def _zero(i, _):
                    def _zg(g, _):
                        acc_v[i, pl.ds(g * 16, 16)] = zero16
                        return None
                    lax.fori_loop(0, _CPW // 16, _zg, None)
                    return None
                lax.fori_loop(0, _OUT_C, _zero, None)

            feat_v, dpt_v, idx_v = feat_b[p], dpt_b[p], idx_b[p]

            def _zbody(z, _):
                @plsc.parallel_loop(0, _CPW // 16, 1, unroll=2)
                def _gbody(g):
                    base = z * _CPW + g * 16
                    iv = idx_v[pl.ds(base, 16)]
                    didxw = iv & m_lo15
                    par = (iv & m_par) != 0
                    fbw = lax.shift_right_logical(iv, 16)
                    wword = plsc.load_gather(dpt_v, [didxw])
                    w_lo = plsc.bitcast(wword << 16, jnp.float32)
                    w_hi = plsc.bitcast(wword & m_hi16, jnp.float32)
                    wvec = jnp.where(par, w_hi, w_lo)
                    for c in range(_OUT_C // 2):
                        word = plsc.load_gather(feat_v, [fbw + c])
                        f_e = plsc.bitcast(word << 16, jnp.float32)
                        f_o = plsc.bitcast(word & m_hi16, jnp.float32)
                        plsc.addupdate(
                            acc_v.at[2 * c, pl.ds(g * 16, 16)], f_e * wvec)
                        plsc.addupdate(
                            acc_v.at[2 * c + 1, pl.ds(g * 16, 16)], f_o * wvec)
                return None
            lax.fori_loop(0, _Z, _zbody, None)

            @pl.when((bn == 5) | (bn == 11))
            def _():
                pltpu.sync_copy(acc_v, out_hbm.at[bn // 6, :, wid])
        return None
    lax.fori_loop(0, 6, _mbody, None)


def _make_perm():
    import numpy as np
    j = np.arange(_NV)
    t = j // (_Z * _CPW)
    z = (j // _CPW) % _Z
    lcol = j % _CPW
    col = t * _CPW + lcol
    y = col // _X
    x = col % _X
    return x * (_Y * _Z) + y * _Z + z


_PERM = _make_perm()


def _make_vox_table():
    import numpy as np
    xs, ys, zs = np.meshgrid(np.arange(_X), np.arange(_Y), np.arange(_Z),
                             indexing='ij')
    coords = np.stack([xs, ys, zs], axis=3).astype(np.float32)
    lb = np.array([-51.2, -51.2, -2.5], dtype=np.float32)
    iv = np.array([0.8, 0.8, 1.0], dtype=np.float32)
    vc = (coords * iv + lb).reshape(-1, 3)[_PERM]
    return np.concatenate([vc.T, np.ones((1, vc.shape[0]), np.float32)],
                          axis=0)


_VOX_T = _make_vox_table()


def _projection_idx(cam2ego, cam_intrinsics, H, W):
    vox_homo_t = jnp.asarray(_VOX_T)
    nv = vox_homo_t.shape[1]

    def _proj(e2c_bn, K_bn):
        cam_t = (e2c_bn @ vox_homo_t)[:3]
        z = cam_t[2]
        valid_z = z > 0.5
        z_safe = jnp.clip(z, 0.1, None)
        norm2_t = cam_t[:2] / z_safe[None, :]
        homo_t = jnp.concatenate([norm2_t, jnp.ones((1, nv), jnp.float32)],
                                 axis=0)
        img_t = (K_bn @ homo_t)[:2]
        fc0 = img_t[0] / _STRIDE
        fc1 = img_t[1] / _STRIDE
        valid = valid_z & (fc0 >= 0) & (fc0 < W) & (fc1 >= 0) & (fc1 < H)
        depth_bin = (z - 1.0).astype(jnp.int32)
        valid = valid & (depth_bin >= 0) & (depth_bin < _D)
        u = jnp.clip(fc0.astype(jnp.int32), 0, W - 1)
        v = jnp.clip(fc1.astype(jnp.int32), 0, H - 1)
        d = jnp.clip(depth_bin, 0, _D - 1)
        pix = v * W + u
        dpart = jnp.where(valid, (pix * _DPSTR + (d >> 1)) | ((d & 1) << 15),
                          _DPZERO)
        return dpart | ((pix * _FPSTR) << 16)

    e2c = jnp.linalg.inv(cam2ego)
    idx = jax.vmap(jax.vmap(_proj))(e2c, cam_intrinsics)
    return idx.reshape(-1, _NV).astype(jnp.int32)


def kernel(img_feats, cam2ego, cam_intrinsics, W_dn, b_dn):
    B, N, C, H, W = img_feats.shape
    BN = B * N
    depth, feat = _dense_stage(img_feats, W_dn, b_dn)

    idx_r = _projection_idx(cam2ego, cam_intrinsics, H, W) \
                .reshape(BN, _NW, _Z * _CPW)

    featp = jax.lax.bitcast_convert_type(
        feat.astype(jnp.bfloat16).reshape(BN, 704, _OUT_C // 2, 2), jnp.int32)
    featp = jnp.pad(featp, ((0, 0), (0, 0), (0, 1))).reshape(BN, _FPSZ)
    dptp = jax.lax.bitcast_convert_type(
        jnp.pad(depth.astype(jnp.bfloat16), ((0, 0), (0, 0), (0, 1)))
           .reshape(BN, 704, _DPSTR - 1, 2), jnp.int32)
    dptp = jnp.pad(dptp, ((0, 0), (0, 0), (0, 1))).reshape(BN, _DPZERO)
    dptp = jnp.pad(dptp, ((0, 0), (0, _DPSZ - _DPZERO)))

    out = _sc_gather(featp, dptp, idx_r)
    bev = out.reshape(B, _OUT_C, _Y, _X)

    depth_out = depth.reshape(B, N, H, W, _D)
    return bev, depth_out
